# baseline (speedup 1.0000x reference)
import jax
import jax.numpy as jnp
import numpy as np
from jax import lax
from jax.experimental import pallas as pl
from jax.experimental.pallas import tpu as pltpu
from jax.experimental.pallas import tpu_sc as plsc

N = 10000
E = 320000
MEM = 128
TIME = 16
EDGE = 16
EPS = 0.1
GAMMA = 0.1

NTILES = 32
CHUNK = 64
CHUNKS_PER_TILE = 158
EPT = CHUNK * CHUNKS_PER_TILE
E_PAD = NTILES * EPT
QW = 160
KVW = 256
N_ACC = 10240
W_ACC = 192
ROWS_PER_TILE = N_ACC // 16
SCALE = 1.0 / np.sqrt(MEM)

_C = [1.0, -1.0 / 2, 1.0 / 24, -1.0 / 720, 1.0 / 40320, -1.0 / 3628800,
      1.0 / 479001600]

BLK = 2000


def _tc_pre_body(x_ref, wenc_ref, benc_ref, wq_ref, bq_ref, wk_ref,
                 bk_ref, wv_ref, bv_ref, we_ref, h_ref, qcat_ref, kv_ref):
    xb = x_ref[...]
    h = jnp.dot(xb, wenc_ref[...], preferred_element_type=jnp.float32)
    h = h + benc_ref[...][None, :]
    q = jnp.dot(h, wq_ref[...], preferred_element_type=jnp.float32) + bq_ref[...][None, :]
    k = jnp.dot(h, wk_ref[...], preferred_element_type=jnp.float32) + bk_ref[...][None, :]
    v = jnp.dot(h, wv_ref[...], preferred_element_type=jnp.float32) + bv_ref[...][None, :]
    q2 = jnp.dot(q, we_ref[...], preferred_element_type=jnp.float32)
    h_ref[...] = h
    qcat_ref[...] = jnp.concatenate([q, q2], axis=1)
    kv_ref[...] = jnp.concatenate([k, v], axis=1).astype(jnp.bfloat16)


def _tc_pre(x, wencT, benc, wqT, bq, wkT, bk, wvT, bv, we):
    nblk = N // BLK
    full = lambda s: pl.BlockSpec(s, lambda i: (0,) * len(s))
    return pl.pallas_call(
        _tc_pre_body,
        grid=(nblk,),
        in_specs=[
            pl.BlockSpec((BLK, MEM + MEM), lambda i: (i, 0)),
            full((MEM + MEM, MEM)), full((MEM,)),
            full((MEM, MEM)), full((MEM,)),
            full((MEM, MEM)), full((MEM,)),
            full((MEM, MEM)), full((MEM,)),
            full((MEM, EDGE + TIME)),
        ],
        out_specs=[
            pl.BlockSpec((BLK, MEM), lambda i: (i, 0)),
            pl.BlockSpec((BLK, QW), lambda i: (i, 0)),
            pl.BlockSpec((BLK, KVW), lambda i: (i, 0)),
        ],
        out_shape=[
            jax.ShapeDtypeStruct((N, MEM), jnp.float32),
            jax.ShapeDtypeStruct((N, QW), jnp.float32),
            jax.ShapeDtypeStruct((N, KVW), jnp.bfloat16),
        ],
    )(x, wencT, benc, wqT, bq, wkT, bk, wvT, bv, we)


def _sc_edge_body(src_hbm, dst_hbm, t_hbm, msg_hbm, lu_hbm, qcat_hbm, kv_hbm,
                  wt_hbm, bt_hbm, out_hbm,
                  acc_sh, srcv, dstv, tv, msgv, qbuf, kvbuf, outbuf, luv,
                  wtv, btv, tebuf, wt_sm, bt_sm, semA, semQ, semK, semS):
    cid = lax.axis_index("c")
    sid = lax.axis_index("s")
    wid = cid * 16 + sid
    tile_base = wid * EPT

    pltpu.sync_copy(wt_hbm, wtv)
    pltpu.sync_copy(bt_hbm, btv)
    pltpu.sync_copy(lu_hbm, luv)

    zrow = jnp.zeros((32,), jnp.bfloat16)
    for r in range(CHUNK):
        for c in range(W_ACC // 32):
            outbuf[r, pl.ds(32 * c, 32)] = zrow
    row0 = sid * ROWS_PER_TILE
    for r in range(ROWS_PER_TILE // CHUNK):
        pltpu.sync_copy(outbuf, acc_sh.at[pl.ds(row0 + r * CHUNK, CHUNK)])
    plsc.subcore_barrier()

    wt16 = wtv[...]
    bt16 = btv[...]
    for f in range(16):
        wt_sm[f] = wt16[f]
        bt_sm[f] = bt16[f]

    pk = lambda a, b: plsc.pack(a, b, format=plsc.PackFormat.INTERLEAVED)
    unpk = lambda ab: plsc.unpack(ab, format=plsc.PackFormat.INTERLEAVED)
    i32 = jnp.int32
    iotav = lax.iota(i32, 16)
    ix = [iotav ^ (1 << b) for b in range(4)]

    def bcast_sum(x):
        for idxv in ix:
            x = x + x.at[idxv].get(mode="promise_in_bounds",
                                   unique_indices=True)
        return x

    def issue_a(ci, s):
        base = tile_base + ci * CHUNK
        cbase = jnp.minimum(base, E - CHUNK)
        pltpu.async_copy(src_hbm.at[pl.ds(base, CHUNK)], srcv.at[s], semA)
        pltpu.async_copy(dst_hbm.at[pl.ds(base, CHUNK)], dstv.at[s], semA)
        pltpu.async_copy(t_hbm.at[pl.ds(cbase, CHUNK)], tv.at[s], semA)
        pltpu.async_copy(msg_hbm.at[pl.ds(cbase, CHUNK)], msgv.at[s], semA)

    def wait_a(s):
        pltpu.make_async_copy(src_hbm.at[pl.ds(0, CHUNK)], srcv.at[s], semA).wait()
        pltpu.make_async_copy(dst_hbm.at[pl.ds(0, CHUNK)], dstv.at[s], semA).wait()
        pltpu.make_async_copy(t_hbm.at[pl.ds(0, CHUNK)], tv.at[s], semA).wait()
        pltpu.make_async_copy(msg_hbm.at[pl.ds(0, CHUNK)], msgv.at[s], semA).wait()

    def issue_b(s):
        pltpu.async_copy(qcat_hbm.at[dstv.at[s]], qbuf.at[s], semQ)
        pltpu.async_copy(kv_hbm.at[srcv.at[s]], kvbuf.at[s], semK)

    def wait_b(s):
        pltpu.make_async_copy(qcat_hbm.at[dstv.at[s]], qbuf.at[s], semQ).wait()
        pltpu.make_async_copy(kv_hbm.at[srcv.at[s]], kvbuf.at[s], semK).wait()

    def wait_scatter(s):
        pltpu.make_async_copy(outbuf, acc_sh.at[dstv.at[s]], semS).wait()

    issue_a(0, 0)
    wait_a(0)
    issue_b(0)

    def compute(ci, s):
        def group_body(g, carry2):
            tg = tv[s, pl.ds(g * 16, 16)]
            srcs = srcv[s, pl.ds(g * 16, 16)]
            lus = plsc.load_gather(luv, [srcs])
            rel = jnp.abs(lus - tg)
            gv = jnp.full((16,), g, i32)
            for f in range(16):
                arg = rel * wt_sm[f] + bt_sm[f]
                u = arg * arg
                te = jnp.full((16,), _C[6], dtype=jnp.float32)
                for c in (5, 4, 3, 2, 1):
                    te = te * u + jnp.float32(_C[c])
                tebuf[g, f, pl.ds(0, 16)] = te * u + jnp.float32(_C[0])
            for h in range(4):
                accs = []
                for j in range(4 * h, 4 * h + 4):
                    e = g * 16 + j
                    te = plsc.load_gather(
                        tebuf, [gv, iotav, jnp.full((16,), j, i32)])
                    q_ = [qbuf[s, e, pl.ds(16 * c, 16)] for c in range(10)]
                    m = msgv[s, e, pl.ds(0, 16)]
                    p = []
                    for c in range(4):
                        ke, ko = unpk(kvbuf[s, e, pl.ds(32 * c, 32)])
                        p += [q_[2 * c] * ke, q_[2 * c + 1] * ko]
                    p += [q_[8] * m, q_[9] * te]
                    while len(p) > 1:
                        p = ([p[i] + p[i + 1]
                              for i in range(0, len(p) - 1, 2)]
                             + ([p[-1]] if len(p) % 2 else []))
                    accs.append(p[0])
                accs = [bcast_sum(a) for a in accs]
                exvs = [jnp.exp(a * jnp.float32(SCALE)) for a in accs]
                for jj in range(4):
                    j = 4 * h + jj
                    e = g * 16 + j
                    exv = exvs[jj]
                    te = plsc.load_gather(
                        tebuf, [gv, iotav, jnp.full((16,), j, i32)])
                    m = msgv[s, e, pl.ds(0, 16)]
                    for c in range(4):
                        ve, vo = unpk(kvbuf[s, e, pl.ds(128 + 32 * c, 32)])
                        outbuf[e, pl.ds(32 * c, 32)] = pk(exv * ve, exv * vo)
                    outbuf[e, pl.ds(128, 32)] = pk(exv * m, exv * te)
                    outbuf[e, pl.ds(160, 32)] = pk(exv, exv)
            return carry2

        lax.fori_loop(0, CHUNK // 16, group_body, 0)

    def pair_body(p, carry):
        for b in (0, 1):
            ci = 2 * p + b
            nci = ci + 1
            nb = 1 - b

            @pl.when(ci > 0)
            def _():
                wait_scatter(nb)

            @pl.when(nci < CHUNKS_PER_TILE)
            def _():
                issue_a(nci, nb)

            wait_b(b)

            @pl.when(nci < CHUNKS_PER_TILE)
            def _():
                wait_a(nb)
                issue_b(nb)

            compute(ci, b)
            pltpu.async_copy(outbuf, acc_sh.at[dstv.at[b]], semS, add=True)
        return carry

    lax.fori_loop(0, CHUNKS_PER_TILE // 2, pair_body, 0)
    wait_scatter(1)
    plsc.subcore_barrier()

    pltpu.sync_copy(acc_sh.at[pl.ds(row0, ROWS_PER_TILE)],
                    out_hbm.at[cid, pl.ds(row0, ROWS_PER_TILE)])


def _sc_edges(src, dst, t, msg, lu, qcat, kv, wt, bt):
    mesh = plsc.VectorSubcoreMesh(core_axis_name="c", subcore_axis_name="s")
    fn = pl.kernel(
        _sc_edge_body,
        mesh=mesh,
        compiler_params=pltpu.CompilerParams(use_tc_tiling_on_sc=False,
                                             needs_layout_passes=False),
        out_type=jax.ShapeDtypeStruct((2, N_ACC, W_ACC), jnp.bfloat16),
        scratch_types=[
            pltpu.VMEM_SHARED((N_ACC, W_ACC), jnp.bfloat16),
            pltpu.VMEM((2, CHUNK), jnp.int32),
            pltpu.VMEM((2, CHUNK), jnp.int32),
            pltpu.VMEM((2, CHUNK), jnp.float32),
            pltpu.VMEM((2, CHUNK, EDGE), jnp.float32),
            pltpu.VMEM((2, CHUNK, QW), jnp.float32),
            pltpu.VMEM((2, CHUNK, KVW), jnp.bfloat16),
            pltpu.VMEM((CHUNK, W_ACC), jnp.bfloat16),
            pltpu.VMEM((N,), jnp.float32),
            pltpu.VMEM((16,), jnp.float32),
            pltpu.VMEM((16,), jnp.float32),
            pltpu.VMEM((CHUNK // 16, 16, 16), jnp.float32),
            pltpu.SMEM((16,), jnp.float32),
            pltpu.SMEM((16,), jnp.float32),
            pltpu.SemaphoreType.DMA,
            pltpu.SemaphoreType.DMA,
            pltpu.SemaphoreType.DMA,
            pltpu.SemaphoreType.DMA,
        ],
    )
    return fn(src, dst, t, msg, lu, qcat, kv, wt, bt)


def _tc_post_body(h_ref, acc_ref, mf_ref, at_ref, banti_ref, out_ref):
    h = h_ref[...]
    accs = acc_ref[0].astype(jnp.float32) + acc_ref[1].astype(jnp.float32)
    phi = jnp.dot(accs, mf_ref[...], preferred_element_type=jnp.float32)
    denom = accs[:, 160:161]
    phi = phi / (denom + jnp.float32(1e-16))
    hh = jnp.tanh(jnp.dot(h, at_ref[...], preferred_element_type=jnp.float32)
                  + phi + banti_ref[...][None, :])
    out_ref[...] = jnp.tanh(h + jnp.float32(EPS) * hh)


def _tc_post(h, acc, mf, aT, banti):
    nblk = N // BLK
    full = lambda s: pl.BlockSpec(s, lambda i: (0,) * len(s))
    return pl.pallas_call(
        _tc_post_body,
        grid=(nblk,),
        in_specs=[
            pl.BlockSpec((BLK, MEM), lambda i: (i, 0)),
            pl.BlockSpec((2, BLK, W_ACC), lambda i: (0, i, 0)),
            full((W_ACC, MEM)),
            full((MEM, MEM)),
            full((MEM,)),
        ],
        out_specs=pl.BlockSpec((BLK, MEM), lambda i: (i, 0)),
        out_shape=jax.ShapeDtypeStruct((N, MEM), jnp.float32),
    )(h, acc, mf, aT, banti)


def kernel(x, last_update, edge_index, t, msg, W_time, b_time, W_enc, b_enc,
           Wq, bq, Wk, bk, Wv, bv, We, W_anti, b_anti):
    f32 = jnp.float32
    p128 = np.empty(MEM, np.int64)
    for g in range(4):
        for i in range(16):
            p128[32 * g + 2 * i] = 32 * g + i
            p128[32 * g + 2 * i + 1] = 32 * g + 16 + i
    h, qcat, kv = _tc_pre(x, W_enc.T, b_enc, Wq.T, bq,
                          Wk.T[:, p128], bk[p128], Wv.T[:, p128], bv[p128],
                          We)

    pad = E_PAD - E
    ar = jnp.arange(pad, dtype=jnp.int32)
    src = jnp.concatenate([edge_index[0], ar % N])
    dst = jnp.concatenate([edge_index[1], N + ar % (N_ACC - N)])
    acc = _sc_edges(src, dst, t, msg, last_update, qcat, kv, W_time[:, 0],
                    b_time)

    sel = np.zeros((W_ACC, MEM), np.float32)
    for g in range(4):
        for i in range(16):
            sel[32 * g + 2 * i, 32 * g + i] = 1.0
            sel[32 * g + 2 * i + 1, 32 * g + 16 + i] = 1.0
    mf = jnp.asarray(sel)
    mf = mf.at[128:160:2, :].set(We[:, :EDGE].T)
    mf = mf.at[129:160:2, :].set(We[:, EDGE:].T)

    aT = W_anti.T - W_anti - f32(GAMMA) * jnp.eye(MEM, dtype=f32)
    return _tc_post(h, acc, mf, aT, b_anti)


# speedup vs baseline: 1.1143x; 1.1143x over previous
import jax
import jax.numpy as jnp
import numpy as np
from jax import lax
from jax.experimental import pallas as pl
from jax.experimental.pallas import tpu as pltpu
from jax.experimental.pallas import tpu_sc as plsc

N = 10000
E = 320000
MEM = 128
TIME = 16
EDGE = 16
EPS = 0.1
GAMMA = 0.1

NTILES = 32
CHUNK = 64
CHUNKS_PER_TILE = 158
EPT = CHUNK * CHUNKS_PER_TILE
E_PAD = NTILES * EPT
QW = 160
KVW = 272
N_ACC = 10240
W_ACC = 192
ROWS_PER_TILE = N_ACC // 16
SCALE = 1.0 / np.sqrt(MEM)

_C = [1.0, -1.0 / 2, 1.0 / 24, -1.0 / 720, 1.0 / 40320, -1.0 / 3628800,
      1.0 / 479001600]

BLK = 2000


def _tc_pre_body(x_ref, lu_ref, wenc_ref, benc_ref, wq_ref, bq_ref, wk_ref,
                 bk_ref, wv_ref, bv_ref, we_ref, h_ref, qcat_ref, kv_ref):
    xb = x_ref[...]
    h = jnp.dot(xb, wenc_ref[...], preferred_element_type=jnp.float32)
    h = h + benc_ref[...][None, :]
    q = jnp.dot(h, wq_ref[...], preferred_element_type=jnp.float32) + bq_ref[...][None, :]
    k = jnp.dot(h, wk_ref[...], preferred_element_type=jnp.float32) + bk_ref[...][None, :]
    v = jnp.dot(h, wv_ref[...], preferred_element_type=jnp.float32) + bv_ref[...][None, :]
    q2 = jnp.dot(q, we_ref[...], preferred_element_type=jnp.float32)
    h_ref[...] = h
    qcat_ref[...] = jnp.concatenate([q, q2], axis=1)
    lupad = jnp.concatenate(
        [lu_ref[...], jnp.zeros((BLK, KVW - 2 * MEM - 1), jnp.float32)],
        axis=1)
    kv_ref[...] = jnp.concatenate([k, v, lupad], axis=1)


def _tc_pre(x, lu, wencT, benc, wqT, bq, wkT, bk, wvT, bv, we):
    nblk = N // BLK
    full = lambda s: pl.BlockSpec(s, lambda i: (0,) * len(s))
    return pl.pallas_call(
        _tc_pre_body,
        grid=(nblk,),
        in_specs=[
            pl.BlockSpec((BLK, MEM + MEM), lambda i: (i, 0)),
            pl.BlockSpec((BLK, 1), lambda i: (i, 0)),
            full((MEM + MEM, MEM)), full((MEM,)),
            full((MEM, MEM)), full((MEM,)),
            full((MEM, MEM)), full((MEM,)),
            full((MEM, MEM)), full((MEM,)),
            full((MEM, EDGE + TIME)),
        ],
        out_specs=[
            pl.BlockSpec((BLK, MEM), lambda i: (i, 0)),
            pl.BlockSpec((BLK, QW), lambda i: (i, 0)),
            pl.BlockSpec((BLK, KVW), lambda i: (i, 0)),
        ],
        out_shape=[
            jax.ShapeDtypeStruct((N, MEM), jnp.float32),
            jax.ShapeDtypeStruct((N, QW), jnp.float32),
            jax.ShapeDtypeStruct((N, KVW), jnp.float32),
        ],
    )(x, lu, wencT, benc, wqT, bq, wkT, bk, wvT, bv, we)


def _sc_edge_body(src_hbm, dst_hbm, t_hbm, msg_hbm, qcat_hbm, kv_hbm,
                  wt_hbm, bt_hbm, out_hbm,
                  acc_sh, srcv, dstv, tv, msgv, qbuf, kvbuf, outbuf,
                  wtv, btv, tebuf, wt_sm, bt_sm, semA, semQ, semK, semS):
    cid = lax.axis_index("c")
    sid = lax.axis_index("s")
    wid = cid * 16 + sid
    tile_base = wid * EPT

    pltpu.sync_copy(wt_hbm, wtv)
    pltpu.sync_copy(bt_hbm, btv)

    zrow = jnp.zeros((32,), jnp.bfloat16)
    for r in range(CHUNK):
        for c in range(W_ACC // 32):
            outbuf[r, pl.ds(32 * c, 32)] = zrow
    row0 = sid * ROWS_PER_TILE
    for r in range(ROWS_PER_TILE // CHUNK):
        pltpu.sync_copy(outbuf, acc_sh.at[pl.ds(row0 + r * CHUNK, CHUNK)])
    plsc.subcore_barrier()

    wt16 = wtv[...]
    bt16 = btv[...]
    for f in range(16):
        wt_sm[f] = wt16[f]
        bt_sm[f] = bt16[f]

    pk = lambda a, b: plsc.pack(a, b, format=plsc.PackFormat.INTERLEAVED)
    unpk = lambda ab: plsc.unpack(ab, format=plsc.PackFormat.INTERLEAVED)
    i32 = jnp.int32
    iotav = lax.iota(i32, 16)
    ix = [iotav ^ (1 << b) for b in range(4)]

    def bcast_sum(x):
        for idxv in ix:
            x = x + x.at[idxv].get(mode="promise_in_bounds",
                                   unique_indices=True)
        return x

    def issue_a(ci, s):
        base = tile_base + ci * CHUNK
        cbase = jnp.minimum(base, E - CHUNK)
        pltpu.async_copy(src_hbm.at[pl.ds(base, CHUNK)], srcv.at[s], semA)
        pltpu.async_copy(dst_hbm.at[pl.ds(base, CHUNK)], dstv.at[s], semA)
        pltpu.async_copy(t_hbm.at[pl.ds(cbase, CHUNK)], tv.at[s], semA)
        pltpu.async_copy(msg_hbm.at[pl.ds(cbase, CHUNK)], msgv.at[s], semA)

    def wait_a(s):
        pltpu.make_async_copy(src_hbm.at[pl.ds(0, CHUNK)], srcv.at[s], semA).wait()
        pltpu.make_async_copy(dst_hbm.at[pl.ds(0, CHUNK)], dstv.at[s], semA).wait()
        pltpu.make_async_copy(t_hbm.at[pl.ds(0, CHUNK)], tv.at[s], semA).wait()
        pltpu.make_async_copy(msg_hbm.at[pl.ds(0, CHUNK)], msgv.at[s], semA).wait()

    def issue_b(s):
        pltpu.async_copy(qcat_hbm.at[dstv.at[s]], qbuf.at[s], semQ)
        pltpu.async_copy(kv_hbm.at[srcv.at[s]], kvbuf.at[s], semK)

    def wait_b(s):
        pltpu.make_async_copy(qcat_hbm.at[dstv.at[s]], qbuf.at[s], semQ).wait()
        pltpu.make_async_copy(kv_hbm.at[srcv.at[s]], kvbuf.at[s], semK).wait()

    def wait_scatter(s):
        pltpu.make_async_copy(outbuf, acc_sh.at[dstv.at[s]], semS).wait()

    issue_a(0, 0)
    wait_a(0)
    issue_b(0)

    def compute(ci, s):
        def group_body(g, carry2):
            tg = tv[s, pl.ds(g * 16, 16)]
            ev = g * 16 + iotav
            lus = plsc.load_gather(
                kvbuf, [jnp.full((16,), s, i32), ev,
                        jnp.full((16,), 2 * MEM, i32)])
            rel = jnp.abs(lus - tg)
            gv = jnp.full((16,), g, i32)
            for f in range(16):
                arg = rel * wt_sm[f] + bt_sm[f]
                u = arg * arg
                te = jnp.full((16,), _C[6], dtype=jnp.float32)
                for c in (5, 4, 3, 2, 1):
                    te = te * u + jnp.float32(_C[c])
                tebuf[g, f, pl.ds(0, 16)] = te * u + jnp.float32(_C[0])
            for h in range(4):
                accs = []
                for j in range(4 * h, 4 * h + 4):
                    e = g * 16 + j
                    te = plsc.load_gather(
                        tebuf, [gv, iotav, jnp.full((16,), j, i32)])
                    q_ = [qbuf[s, e, pl.ds(16 * c, 16)] for c in range(10)]
                    m = msgv[s, e, pl.ds(0, 16)]
                    p = [q_[c] * kvbuf[s, e, pl.ds(16 * c, 16)]
                         for c in range(8)]
                    p += [q_[8] * m, q_[9] * te]
                    while len(p) > 1:
                        p = ([p[i] + p[i + 1]
                              for i in range(0, len(p) - 1, 2)]
                             + ([p[-1]] if len(p) % 2 else []))
                    accs.append(p[0])
                accs = [bcast_sum(a) for a in accs]
                exvs = [jnp.exp(a * jnp.float32(SCALE)) for a in accs]
                for jj in range(4):
                    j = 4 * h + jj
                    e = g * 16 + j
                    exv = exvs[jj]
                    te = plsc.load_gather(
                        tebuf, [gv, iotav, jnp.full((16,), j, i32)])
                    m = msgv[s, e, pl.ds(0, 16)]
                    vo = [exv * kvbuf[s, e, pl.ds(128 + 16 * c, 16)]
                          for c in range(8)]
                    outbuf[e, pl.ds(0, 32)] = pk(vo[0], vo[1])
                    outbuf[e, pl.ds(32, 32)] = pk(vo[2], vo[3])
                    outbuf[e, pl.ds(64, 32)] = pk(vo[4], vo[5])
                    outbuf[e, pl.ds(96, 32)] = pk(vo[6], vo[7])
                    outbuf[e, pl.ds(128, 32)] = pk(exv * m, exv * te)
                    outbuf[e, pl.ds(160, 32)] = pk(exv, exv)
            return carry2

        lax.fori_loop(0, CHUNK // 16, group_body, 0)

    def pair_body(p, carry):
        for b in (0, 1):
            ci = 2 * p + b
            nci = ci + 1
            nb = 1 - b

            @pl.when(ci > 0)
            def _():
                wait_scatter(nb)

            @pl.when(nci < CHUNKS_PER_TILE)
            def _():
                issue_a(nci, nb)

            wait_b(b)

            @pl.when(nci < CHUNKS_PER_TILE)
            def _():
                wait_a(nb)
                issue_b(nb)

            compute(ci, b)
            pltpu.async_copy(outbuf, acc_sh.at[dstv.at[b]], semS, add=True)
        return carry

    lax.fori_loop(0, CHUNKS_PER_TILE // 2, pair_body, 0)
    wait_scatter(1)
    plsc.subcore_barrier()

    pltpu.sync_copy(acc_sh.at[pl.ds(row0, ROWS_PER_TILE)],
                    out_hbm.at[cid, pl.ds(row0, ROWS_PER_TILE)])


def _sc_edges(src, dst, t, msg, qcat, kv, wt, bt):
    mesh = plsc.VectorSubcoreMesh(core_axis_name="c", subcore_axis_name="s")
    fn = pl.kernel(
        _sc_edge_body,
        mesh=mesh,
        compiler_params=pltpu.CompilerParams(use_tc_tiling_on_sc=False,
                                             needs_layout_passes=False),
        out_type=jax.ShapeDtypeStruct((2, N_ACC, W_ACC), jnp.bfloat16),
        scratch_types=[
            pltpu.VMEM_SHARED((N_ACC, W_ACC), jnp.bfloat16),
            pltpu.VMEM((2, CHUNK), jnp.int32),
            pltpu.VMEM((2, CHUNK), jnp.int32),
            pltpu.VMEM((2, CHUNK), jnp.float32),
            pltpu.VMEM((2, CHUNK, EDGE), jnp.float32),
            pltpu.VMEM((2, CHUNK, QW), jnp.float32),
            pltpu.VMEM((2, CHUNK, KVW), jnp.float32),
            pltpu.VMEM((CHUNK, W_ACC), jnp.bfloat16),
            pltpu.VMEM((16,), jnp.float32),
            pltpu.VMEM((16,), jnp.float32),
            pltpu.VMEM((CHUNK // 16, 16, 16), jnp.float32),
            pltpu.SMEM((16,), jnp.float32),
            pltpu.SMEM((16,), jnp.float32),
            pltpu.SemaphoreType.DMA,
            pltpu.SemaphoreType.DMA,
            pltpu.SemaphoreType.DMA,
            pltpu.SemaphoreType.DMA,
        ],
    )
    return fn(src, dst, t, msg, qcat, kv, wt, bt)


def _tc_post_body(h_ref, acc_ref, mf_ref, at_ref, banti_ref, out_ref):
    h = h_ref[...]
    accs = acc_ref[0].astype(jnp.float32) + acc_ref[1].astype(jnp.float32)
    phi = jnp.dot(accs, mf_ref[...], preferred_element_type=jnp.float32)
    denom = accs[:, 160:161]
    phi = phi / (denom + jnp.float32(1e-16))
    hh = jnp.tanh(jnp.dot(h, at_ref[...], preferred_element_type=jnp.float32)
                  + phi + banti_ref[...][None, :])
    out_ref[...] = jnp.tanh(h + jnp.float32(EPS) * hh)


def _tc_post(h, acc, mf, aT, banti):
    nblk = N // BLK
    full = lambda s: pl.BlockSpec(s, lambda i: (0,) * len(s))
    return pl.pallas_call(
        _tc_post_body,
        grid=(nblk,),
        in_specs=[
            pl.BlockSpec((BLK, MEM), lambda i: (i, 0)),
            pl.BlockSpec((2, BLK, W_ACC), lambda i: (0, i, 0)),
            full((W_ACC, MEM)),
            full((MEM, MEM)),
            full((MEM,)),
        ],
        out_specs=pl.BlockSpec((BLK, MEM), lambda i: (i, 0)),
        out_shape=jax.ShapeDtypeStruct((N, MEM), jnp.float32),
    )(h, acc, mf, aT, banti)


def kernel(x, last_update, edge_index, t, msg, W_time, b_time, W_enc, b_enc,
           Wq, bq, Wk, bk, Wv, bv, We, W_anti, b_anti):
    f32 = jnp.float32
    h, qcat, kv = _tc_pre(x, last_update[:, None], W_enc.T, b_enc, Wq.T, bq,
                          Wk.T, bk, Wv.T, bv, We)

    pad = E_PAD - E
    ar = jnp.arange(pad, dtype=jnp.int32)
    src = jnp.concatenate([edge_index[0], ar % N])
    dst = jnp.concatenate([edge_index[1], N + ar % (N_ACC - N)])
    acc = _sc_edges(src, dst, t, msg, qcat, kv, W_time[:, 0], b_time)

    sel = np.zeros((W_ACC, MEM), np.float32)
    for g in range(4):
        for i in range(16):
            sel[32 * g + 2 * i, 32 * g + i] = 1.0
            sel[32 * g + 2 * i + 1, 32 * g + 16 + i] = 1.0
    mf = jnp.asarray(sel)
    mf = mf.at[128:160:2, :].set(We[:, :EDGE].T)
    mf = mf.at[129:160:2, :].set(We[:, EDGE:].T)

    aT = W_anti.T - W_anti - f32(GAMMA) * jnp.eye(MEM, dtype=f32)
    return _tc_post(h, acc, mf, aT, b_anti)
